# baseline (speedup 1.0000x reference)
import jax
import jax.numpy as jnp
from jax import lax
from jax.experimental import pallas as pl
from jax.experimental.pallas import tpu as pltpu
from jax.experimental.pallas import tpu_sc as plsc

N = 10000
NP = 10240
E = 320000
D = 128
K = 3
CH = 64
DCH = 128
E_PAD = 327680
NCHUNK = E_PAD // CH
NC, NS = 2, 16
CPS = NCHUNK // NC
CPT = CPS // NS
NPHASE = 4
PPT = CPT // NPHASE
DEG_ROWS = K * E_PAD // DCH
DEG_RPT = DEG_ROWS // (NC * NS)
RPT = NP // NS
FRPT = 624
FTAIL = N - NS * FRPT


def _sc_mesh():
    return plsc.VectorSubcoreMesh(core_axis_name="c", subcore_axis_name="s")


def _deg_body(dd_hbm, zero_hbm, out_hbm, idx_v, ones_v, dsem, acc_sh):
    c = lax.axis_index("c")
    s = lax.axis_index("s")
    wid = c * NS + s

    @pl.when(s == 0)
    def _init():
        pltpu.sync_copy(zero_hbm, acc_sh)

    for i in range(8):
        ones_v[pl.ds(i * 16, 16)] = jnp.ones((16,), jnp.float32)
    pltpu.sync_copy(dd_hbm.at[pl.ds(wid * DEG_RPT, DEG_RPT)], idx_v)
    plsc.subcore_barrier()

    W = 16
    for b in range(W):
        pltpu.async_copy(ones_v, acc_sh.at[idx_v.at[b]], dsem, add=True)

    def step(j, carry):
        pltpu.make_async_copy(ones_v, acc_sh.at[idx_v.at[0]], dsem).wait()
        pltpu.async_copy(ones_v, acc_sh.at[idx_v.at[j + W]], dsem, add=True)
        return carry

    lax.fori_loop(0, DEG_RPT - W, step, 0)
    for b in range(W):
        pltpu.make_async_copy(ones_v, acc_sh.at[idx_v.at[0]], dsem).wait()
    plsc.subcore_barrier()

    @pl.when(s == 0)
    def _flush():
        pltpu.sync_copy(acc_sh, out_hbm.at[c])


def _degrees(dd, zero_deg):
    return pl.kernel(
        _deg_body,
        out_type=jax.ShapeDtypeStruct((NC, K * NP), jnp.float32),
        mesh=_sc_mesh(),
        scratch_types=[
            pltpu.VMEM((DEG_RPT, DCH), jnp.int32),
            pltpu.VMEM((DCH,), jnp.float32),
            pltpu.SemaphoreType.DMA,
            pltpu.VMEM_SHARED((K * NP,), jnp.float32),
        ],
    )(dd, zero_deg)


def _scale_body(x_ref, w_ref, deg_ref, g0_ref, g1_ref, g2_ref, dinv_ref):
    deg = deg_ref[...]
    degsum = deg[:, :K] + deg[:, K:] + 1.0
    dinv = lax.rsqrt(jnp.maximum(degsum, 1e-12))
    h = jnp.dot(x_ref[...], w_ref[...],
                preferred_element_type=jnp.float32)
    for k, g_ref in enumerate((g0_ref, g1_ref, g2_ref)):
        g_ref[...] = h[:, k * D:(k + 1) * D] * dinv[:, k][:, None]
    dinv_ref[...] = dinv


def _scale(x, wcat, deg_t):
    B = 1000
    return pl.pallas_call(
        _scale_body,
        grid=(N // B,),
        in_specs=[
            pl.BlockSpec((B, D), lambda i: (i, 0)),
            pl.BlockSpec((D, K * D), lambda i: (0, 0)),
            pl.BlockSpec((B, NC * K), lambda i: (i, 0)),
        ],
        out_specs=[pl.BlockSpec((B, D), lambda i: (i, 0))] * K + [
            pl.BlockSpec((B, K), lambda i: (i, 0)),
        ],
        out_shape=[jax.ShapeDtypeStruct((N, D), jnp.float32)] * K + [
            jax.ShapeDtypeStruct((N, K), jnp.float32),
        ],
    )(x, wcat, deg_t)


NBUF = 4


def _edge_body(g0_hbm, g1_hbm, g2_hbm, gsrc_hbm, sdst_hbm, zero_hbm, p_hbm,
               src_v, dst_v, r0, r1, r2, r3, g0, g1, g2, g3,
               s0, s1, s2, s3, acc_sh):
    rows = (r0, r1, r2, r3)
    gsems = (g0, g1, g2, g3)
    ssems = (s0, s1, s2, s3)
    c = lax.axis_index("c")
    s = lax.axis_index("s")
    for k in range(K):
        gk_hbm = (g0_hbm, g1_hbm, g2_hbm)[k]
        pltpu.sync_copy(zero_hbm.at[pl.ds(s * RPT, RPT)],
                        acc_sh.at[pl.ds(s * RPT, RPT)])
        for p in range(NPHASE):
            base = c * CPS + s * CPT + p * PPT
            pltpu.sync_copy(gsrc_hbm.at[k].at[pl.ds(base, PPT)], src_v)
            pltpu.sync_copy(sdst_hbm.at[k].at[pl.ds(base, PPT)], dst_v)
            if p == 0:
                plsc.subcore_barrier()

            for b in range(NBUF - 1):
                pltpu.async_copy(gk_hbm.at[src_v.at[b]], rows[b], gsems[b])

            def rnd(jj, carry):
                for b in range(NBUF):
                    j = NBUF * jj + b
                    pltpu.make_async_copy(
                        gk_hbm.at[src_v.at[j]], rows[b], gsems[b]).wait()
                    pltpu.async_copy(
                        rows[b], acc_sh.at[dst_v.at[j]], ssems[b], add=True)
                    nxt = j + NBUF - 1
                    bn = (b + NBUF - 1) % NBUF

                    @pl.when(jnp.logical_and(j >= 1, nxt < PPT))
                    def _wait_prev():
                        pltpu.make_async_copy(
                            rows[bn], acc_sh.at[dst_v.at[j]],
                            ssems[bn]).wait()

                    @pl.when(nxt < PPT)
                    def _issue():
                        pltpu.async_copy(
                            gk_hbm.at[src_v.at[nxt]], rows[bn], gsems[bn])
                return carry

            lax.fori_loop(0, PPT // NBUF, rnd, 0)
            for b in range(NBUF):
                pltpu.make_async_copy(
                    rows[b], acc_sh.at[dst_v.at[0]], ssems[b]).wait()
        plsc.subcore_barrier()
        pltpu.sync_copy(acc_sh.at[pl.ds(s * FRPT, FRPT)],
                        p_hbm.at[c].at[k].at[pl.ds(s * FRPT, FRPT)])

        @pl.when(s == NS - 1)
        def _tail():
            pltpu.sync_copy(
                acc_sh.at[pl.ds(NS * FRPT, FTAIL)],
                p_hbm.at[c].at[k].at[pl.ds(NS * FRPT, FTAIL)])
        plsc.subcore_barrier()


def _edges(g0, g1, g2, gsrc, sdst, zero_rows):
    return pl.kernel(
        _edge_body,
        out_type=jax.ShapeDtypeStruct((NC, K, N, D), jnp.float32),
        mesh=_sc_mesh(),
        scratch_types=[
            pltpu.VMEM((PPT, CH), jnp.int32),
            pltpu.VMEM((PPT, CH), jnp.int32),
            pltpu.VMEM((CH, D), jnp.float32),
            pltpu.VMEM((CH, D), jnp.float32),
            pltpu.VMEM((CH, D), jnp.float32),
            pltpu.VMEM((CH, D), jnp.float32),
            pltpu.SemaphoreType.DMA,
            pltpu.SemaphoreType.DMA,
            pltpu.SemaphoreType.DMA,
            pltpu.SemaphoreType.DMA,
            pltpu.SemaphoreType.DMA,
            pltpu.SemaphoreType.DMA,
            pltpu.SemaphoreType.DMA,
            pltpu.SemaphoreType.DMA,
            pltpu.VMEM_SHARED((NP, D), jnp.float32),
        ],
    )(g0, g1, g2, gsrc, sdst, zero_rows)


def _combine_body(p_ref, g0_ref, g1_ref, g2_ref, dinv_ref, out_ref):
    acc = jnp.zeros_like(out_ref[...])
    for k, g_ref in enumerate((g0_ref, g1_ref, g2_ref)):
        acc = acc + dinv_ref[:, k][:, None] * (
            p_ref[0, k] + p_ref[1, k] + g_ref[...])
    out_ref[...] = acc


def _combine(p, g0, g1, g2, dinv):
    B = 1000
    return pl.pallas_call(
        _combine_body,
        grid=(N // B,),
        in_specs=[
            pl.BlockSpec((NC, K, B, D), lambda i: (0, 0, i, 0)),
        ] + [pl.BlockSpec((B, D), lambda i: (i, 0))] * K + [
            pl.BlockSpec((B, K), lambda i: (i, 0)),
        ],
        out_specs=pl.BlockSpec((B, D), lambda i: (i, 0)),
        out_shape=jax.ShapeDtypeStruct((N, D), jnp.float32),
    )(p, g0, g1, g2, dinv)


def kernel(x, adj0, adj1, adj2, W0, W1, W2):
    adjs = [jnp.asarray(a, jnp.int32) for a in (adj0, adj1, adj2)]
    pad = E_PAD - E
    pad_src = (jnp.arange(pad, dtype=jnp.int32) * 977) % N
    pad_dst = N + (jnp.arange(pad, dtype=jnp.int32) % (NP - N))

    gsrc = jnp.concatenate(
        [adjs[0][0], pad_src, adjs[1][0], pad_src, adjs[2][0], pad_src]
    ).reshape(K, NCHUNK, CH)
    dcat = jnp.concatenate(
        [adjs[0][1], pad_dst, adjs[1][1], pad_dst, adjs[2][1], pad_dst])
    sdst = dcat.reshape(K, NCHUNK, CH)
    doffs = jnp.repeat(jnp.arange(K, dtype=jnp.int32) * NP, E_PAD)
    dd = (dcat + doffs).reshape(DEG_ROWS, DCH)

    zero_deg = jnp.zeros((K * NP,), jnp.float32)
    zero_rows = jnp.zeros((NP, D), jnp.float32)
    wcat = jnp.concatenate([W0, W1, W2], axis=1)

    degs = _degrees(dd, zero_deg)
    deg_t = (degs.reshape(NC, K, NP)[:, :, :N]
             .transpose(2, 0, 1).reshape(N, NC * K))
    g0, g1, g2, dinv = _scale(x, wcat, deg_t)
    p = _edges(g0, g1, g2, gsrc, sdst, zero_rows)
    return _combine(p, g0, g1, g2, dinv)


# speedup vs baseline: 1.0008x; 1.0008x over previous
import jax
import jax.numpy as jnp
from jax import lax
from jax.experimental import pallas as pl
from jax.experimental.pallas import tpu as pltpu
from jax.experimental.pallas import tpu_sc as plsc

N = 10000
NP = 10240
E = 320000
D = 128
K = 3
CH = 64
DCH = 128
E_PAD = 327680
NCHUNK = E_PAD // CH
NC, NS = 2, 16
CPS = NCHUNK // NC
CPT = CPS // NS
NPHASE = 4
PPT = CPT // NPHASE
DEG_ROWS = K * E_PAD // DCH
DEG_RPT = DEG_ROWS // (NC * NS)
RPT = NP // NS
FRPT = 624
FTAIL = N - NS * FRPT


def _sc_mesh():
    return plsc.VectorSubcoreMesh(core_axis_name="c", subcore_axis_name="s")


def _deg_body(dd_hbm, zero_hbm, out_hbm, idx_v, ones_v, dsem, acc_sh):
    c = lax.axis_index("c")
    s = lax.axis_index("s")
    wid = c * NS + s

    @pl.when(s == 0)
    def _init():
        pltpu.sync_copy(zero_hbm, acc_sh)

    for i in range(8):
        ones_v[pl.ds(i * 16, 16)] = jnp.ones((16,), jnp.float32)
    pltpu.sync_copy(dd_hbm.at[pl.ds(wid * DEG_RPT, DEG_RPT)], idx_v)
    plsc.subcore_barrier()

    W = 16
    for b in range(W):
        pltpu.async_copy(ones_v, acc_sh.at[idx_v.at[b]], dsem, add=True)

    def step(j, carry):
        pltpu.make_async_copy(ones_v, acc_sh.at[idx_v.at[0]], dsem).wait()
        pltpu.async_copy(ones_v, acc_sh.at[idx_v.at[j + W]], dsem, add=True)
        return carry

    lax.fori_loop(0, DEG_RPT - W, step, 0)
    for b in range(W):
        pltpu.make_async_copy(ones_v, acc_sh.at[idx_v.at[0]], dsem).wait()
    plsc.subcore_barrier()

    @pl.when(s == 0)
    def _flush():
        pltpu.sync_copy(acc_sh, out_hbm.at[c])


def _degrees(dd, zero_deg):
    return pl.kernel(
        _deg_body,
        out_type=jax.ShapeDtypeStruct((NC, K * NP), jnp.float32),
        mesh=_sc_mesh(),
        scratch_types=[
            pltpu.VMEM((DEG_RPT, DCH), jnp.int32),
            pltpu.VMEM((DCH,), jnp.float32),
            pltpu.SemaphoreType.DMA,
            pltpu.VMEM_SHARED((K * NP,), jnp.float32),
        ],
    )(dd, zero_deg)


def _scale_body(x_ref, w_ref, deg_ref, g0_ref, g1_ref, g2_ref, dinv_ref):
    deg = deg_ref[...]
    degsum = deg[:, :K] + deg[:, K:] + 1.0
    dinv = lax.rsqrt(jnp.maximum(degsum, 1e-12))
    h = jnp.dot(x_ref[...], w_ref[...],
                preferred_element_type=jnp.float32)
    for k, g_ref in enumerate((g0_ref, g1_ref, g2_ref)):
        g_ref[...] = h[:, k * D:(k + 1) * D] * dinv[:, k][:, None]
    dinv_ref[...] = dinv


def _scale(x, wcat, deg_t):
    B = 1000
    return pl.pallas_call(
        _scale_body,
        grid=(N // B,),
        in_specs=[
            pl.BlockSpec((B, D), lambda i: (i, 0)),
            pl.BlockSpec((D, K * D), lambda i: (0, 0)),
            pl.BlockSpec((B, NC * K), lambda i: (i, 0)),
        ],
        out_specs=[pl.BlockSpec((B, D), lambda i: (i, 0))] * K + [
            pl.BlockSpec((B, K), lambda i: (i, 0)),
        ],
        out_shape=[jax.ShapeDtypeStruct((N, D), jnp.float32)] * K + [
            jax.ShapeDtypeStruct((N, K), jnp.float32),
        ],
    )(x, wcat, deg_t)


NBUF = 4


def _edge_body(g0_hbm, g1_hbm, g2_hbm, gsrc_hbm, sdst_hbm, zero_hbm, p_hbm,
               src_v, dst_v, r0, r1, r2, r3, g0, g1, g2, g3,
               s0, s1, s2, s3, acc_sh):
    rows = (r0, r1, r2, r3)
    gsems = (g0, g1, g2, g3)
    ssems = (s0, s1, s2, s3)
    c = lax.axis_index("c")
    s = lax.axis_index("s")
    for k in range(K):
        gk_hbm = (g0_hbm, g1_hbm, g2_hbm)[k]
        pltpu.sync_copy(zero_hbm.at[pl.ds(s * RPT, RPT)],
                        acc_sh.at[pl.ds(s * RPT, RPT)])
        for p in range(NPHASE):
            base = c * CPS + s * CPT + p * PPT
            pltpu.sync_copy(gsrc_hbm.at[k].at[pl.ds(base, PPT)], src_v)
            pltpu.sync_copy(sdst_hbm.at[k].at[pl.ds(base, PPT)], dst_v)
            if p == 0:
                plsc.subcore_barrier()

            for b in range(NBUF - 1):
                pltpu.async_copy(gk_hbm.at[src_v.at[b]], rows[b], gsems[b])

            def rnd(jj, carry):
                for b in range(NBUF):
                    j = NBUF * jj + b
                    pltpu.make_async_copy(
                        gk_hbm.at[src_v.at[j]], rows[b], gsems[b]).wait()
                    pltpu.async_copy(
                        rows[b], acc_sh.at[dst_v.at[j]], ssems[b], add=True)
                    nxt = j + NBUF - 1
                    bn = (b + NBUF - 1) % NBUF

                    @pl.when(jnp.logical_and(j >= 1, nxt < PPT))
                    def _wait_prev():
                        pltpu.make_async_copy(
                            rows[bn], acc_sh.at[dst_v.at[j]],
                            ssems[bn]).wait()

                    @pl.when(nxt < PPT)
                    def _issue():
                        pltpu.async_copy(
                            gk_hbm.at[src_v.at[nxt]], rows[bn], gsems[bn])
                return carry

            lax.fori_loop(0, PPT // NBUF, rnd, 0)
            for b in range(NBUF):
                pltpu.make_async_copy(
                    rows[b], acc_sh.at[dst_v.at[0]], ssems[b]).wait()
        plsc.subcore_barrier()
        pltpu.sync_copy(acc_sh.at[pl.ds(s * FRPT, FRPT)],
                        p_hbm.at[c].at[k].at[pl.ds(s * FRPT, FRPT)])

        @pl.when(s == NS - 1)
        def _tail():
            pltpu.sync_copy(
                acc_sh.at[pl.ds(NS * FRPT, FTAIL)],
                p_hbm.at[c].at[k].at[pl.ds(NS * FRPT, FTAIL)])
        plsc.subcore_barrier()


def _edges(g0, g1, g2, gsrc, sdst, zero_rows):
    return pl.kernel(
        _edge_body,
        out_type=jax.ShapeDtypeStruct((NC, K, N, D), jnp.float32),
        mesh=_sc_mesh(),
        scratch_types=[
            pltpu.VMEM((PPT, CH), jnp.int32),
            pltpu.VMEM((PPT, CH), jnp.int32),
            pltpu.VMEM((CH, D), jnp.float32),
            pltpu.VMEM((CH, D), jnp.float32),
            pltpu.VMEM((CH, D), jnp.float32),
            pltpu.VMEM((CH, D), jnp.float32),
            pltpu.SemaphoreType.DMA,
            pltpu.SemaphoreType.DMA,
            pltpu.SemaphoreType.DMA,
            pltpu.SemaphoreType.DMA,
            pltpu.SemaphoreType.DMA,
            pltpu.SemaphoreType.DMA,
            pltpu.SemaphoreType.DMA,
            pltpu.SemaphoreType.DMA,
            pltpu.VMEM_SHARED((NP, D), jnp.float32),
        ],
    )(g0, g1, g2, gsrc, sdst, zero_rows)


def _combine_body(p_ref, g0_ref, g1_ref, g2_ref, dinv_ref, out_ref):
    acc = jnp.zeros_like(out_ref[...])
    for k, g_ref in enumerate((g0_ref, g1_ref, g2_ref)):
        acc = acc + dinv_ref[:, k][:, None] * (
            p_ref[0, k] + p_ref[1, k] + g_ref[...])
    out_ref[...] = acc


def _combine(p, g0, g1, g2, dinv):
    B = 1000
    return pl.pallas_call(
        _combine_body,
        grid=(N // B,),
        in_specs=[
            pl.BlockSpec((NC, K, B, D), lambda i: (0, 0, i, 0)),
        ] + [pl.BlockSpec((B, D), lambda i: (i, 0))] * K + [
            pl.BlockSpec((B, K), lambda i: (i, 0)),
        ],
        out_specs=pl.BlockSpec((B, D), lambda i: (i, 0)),
        out_shape=jax.ShapeDtypeStruct((N, D), jnp.float32),
    )(p, g0, g1, g2, dinv)


def kernel(x, adj0, adj1, adj2, W0, W1, W2):
    adjs = [jnp.asarray(a, jnp.int32) for a in (adj0, adj1, adj2)]
    pad = E_PAD - E
    pad_src = (jnp.arange(pad, dtype=jnp.int32) * 977) % N
    pad_dst = N + (jnp.arange(pad, dtype=jnp.int32) % (NP - N))

    gsrc = jnp.concatenate(
        [adjs[0][0], pad_src, adjs[1][0], pad_src, adjs[2][0], pad_src]
    ).reshape(K, NCHUNK, CH)
    dcat = jnp.concatenate(
        [adjs[0][1], pad_dst, adjs[1][1], pad_dst, adjs[2][1], pad_dst])
    sdst = dcat.reshape(K, NCHUNK, CH)
    doffs = (jnp.arange(K, dtype=jnp.int32) * NP)[:, None]
    dd = (dcat.reshape(K, E_PAD) + doffs).reshape(DEG_ROWS, DCH)

    zero_deg = jnp.zeros((K * NP,), jnp.float32)
    zero_rows = jnp.zeros((NP, D), jnp.float32)
    wcat = jnp.concatenate([W0, W1, W2], axis=1)

    degs = _degrees(dd, zero_deg)
    deg_t = (degs.reshape(NC, K, NP)[:, :, :N]
             .transpose(2, 0, 1).reshape(N, NC * K))
    g0, g1, g2, dinv = _scale(x, wcat, deg_t)
    p = _edges(g0, g1, g2, gsrc, sdst, zero_rows)
    return _combine(p, g0, g1, g2, dinv)


# speedup vs baseline: 1.1114x; 1.1105x over previous
import jax
import jax.numpy as jnp
from jax import lax
from jax.experimental import pallas as pl
from jax.experimental.pallas import tpu as pltpu
from jax.experimental.pallas import tpu_sc as plsc

N = 10000
NP = 10240
E = 320000
D = 128
K = 3
CH = 64
DCH = 128
E_PAD = 327680
NCHUNK = E_PAD // CH
NC, NS = 2, 16
CPS = NCHUNK // NC
CPT = CPS // NS
NPHASE = 4
PPT = CPT // NPHASE
DEG_ROWS = K * E_PAD // DCH
DEG_RPT = DEG_ROWS // (NC * NS)
RPT = NP // NS
FRPT = 624
FTAIL = N - NS * FRPT


def _sc_mesh():
    return plsc.VectorSubcoreMesh(core_axis_name="c", subcore_axis_name="s")


def _deg_body(dd_hbm, zero_hbm, out_hbm, idx_v, ones_v, dsem, acc_sh):
    c = lax.axis_index("c")
    s = lax.axis_index("s")
    wid = c * NS + s

    @pl.when(s == 0)
    def _init():
        pltpu.sync_copy(zero_hbm, acc_sh)

    for i in range(8):
        ones_v[pl.ds(i * 16, 16)] = jnp.ones((16,), jnp.float32)
    pltpu.sync_copy(dd_hbm.at[pl.ds(wid * DEG_RPT, DEG_RPT)], idx_v)
    plsc.subcore_barrier()

    W = 16
    for b in range(W):
        pltpu.async_copy(ones_v, acc_sh.at[idx_v.at[b]], dsem, add=True)

    def step(j, carry):
        pltpu.make_async_copy(ones_v, acc_sh.at[idx_v.at[0]], dsem).wait()
        pltpu.async_copy(ones_v, acc_sh.at[idx_v.at[j + W]], dsem, add=True)
        return carry

    lax.fori_loop(0, DEG_RPT - W, step, 0)
    for b in range(W):
        pltpu.make_async_copy(ones_v, acc_sh.at[idx_v.at[0]], dsem).wait()
    plsc.subcore_barrier()

    @pl.when(s == 0)
    def _flush():
        pltpu.sync_copy(acc_sh, out_hbm.at[c])


def _degrees(dd, zero_deg):
    return pl.kernel(
        _deg_body,
        out_type=jax.ShapeDtypeStruct((NC, K * NP), jnp.float32),
        mesh=_sc_mesh(),
        scratch_types=[
            pltpu.VMEM((DEG_RPT, DCH), jnp.int32),
            pltpu.VMEM((DCH,), jnp.float32),
            pltpu.SemaphoreType.DMA,
            pltpu.VMEM_SHARED((K * NP,), jnp.float32),
        ],
    )(dd, zero_deg)


def _scale_body(x_ref, w_ref, deg_ref, g0_ref, g1_ref, g2_ref, dinv_ref):
    deg = deg_ref[...]
    degsum = deg[:, :K] + deg[:, K:] + 1.0
    dinv = lax.rsqrt(jnp.maximum(degsum, 1e-12))
    h = jnp.dot(x_ref[...], w_ref[...],
                preferred_element_type=jnp.float32)
    for k, g_ref in enumerate((g0_ref, g1_ref, g2_ref)):
        g_ref[...] = h[:, k * D:(k + 1) * D] * dinv[:, k][:, None]
    dinv_ref[...] = dinv


def _scale(x, wcat, deg_t):
    B = 1000
    return pl.pallas_call(
        _scale_body,
        grid=(N // B,),
        in_specs=[
            pl.BlockSpec((B, D), lambda i: (i, 0)),
            pl.BlockSpec((D, K * D), lambda i: (0, 0)),
            pl.BlockSpec((B, NC * K), lambda i: (i, 0)),
        ],
        out_specs=[pl.BlockSpec((B, D), lambda i: (i, 0))] * K + [
            pl.BlockSpec((B, K), lambda i: (i, 0)),
        ],
        out_shape=[jax.ShapeDtypeStruct((N, D), jnp.float32)] * K + [
            jax.ShapeDtypeStruct((N, K), jnp.float32),
        ],
    )(x, wcat, deg_t)


NBUF = 4


def _edge_body(g0_hbm, g1_hbm, g2_hbm, gsrc_hbm, sdst_hbm, zero_hbm, p_hbm,
               src_v, dst_v, r0, r1, r2, r3, g0, g1, g2, g3,
               s0, s1, s2, s3, acc_sh):
    rows = (r0, r1, r2, r3)
    gsems = (g0, g1, g2, g3)
    ssems = (s0, s1, s2, s3)
    c = lax.axis_index("c")
    s = lax.axis_index("s")
    for k in range(K):
        gk_hbm = (g0_hbm, g1_hbm, g2_hbm)[k]
        pltpu.sync_copy(zero_hbm.at[pl.ds(s * RPT, RPT)],
                        acc_sh.at[pl.ds(s * RPT, RPT)])
        for p in range(NPHASE):
            base = c * CPS + s * CPT + p * PPT
            pltpu.sync_copy(gsrc_hbm.at[k].at[pl.ds(base, PPT)], src_v)
            pltpu.sync_copy(sdst_hbm.at[k].at[pl.ds(base, PPT)], dst_v)
            if p == 0:
                plsc.subcore_barrier()

            for b in range(NBUF - 1):
                pltpu.async_copy(gk_hbm.at[src_v.at[b]], rows[b], gsems[b])

            def rnd(jj, carry):
                for b in range(NBUF):
                    j = NBUF * jj + b
                    pltpu.make_async_copy(
                        gk_hbm.at[src_v.at[j]], rows[b], gsems[b]).wait()
                    pltpu.async_copy(
                        rows[b], acc_sh.at[dst_v.at[j]], ssems[b], add=True)
                    nxt = j + NBUF - 1
                    bn = (b + NBUF - 1) % NBUF

                    @pl.when(jnp.logical_and(j >= 1, nxt < PPT))
                    def _wait_prev():
                        pltpu.make_async_copy(
                            rows[bn], acc_sh.at[dst_v.at[j]],
                            ssems[bn]).wait()

                    @pl.when(nxt < PPT)
                    def _issue():
                        pltpu.async_copy(
                            gk_hbm.at[src_v.at[nxt]], rows[bn], gsems[bn])
                return carry

            lax.fori_loop(0, PPT // NBUF, rnd, 0)
            for b in range(NBUF):
                pltpu.make_async_copy(
                    rows[b], acc_sh.at[dst_v.at[0]], ssems[b]).wait()
        plsc.subcore_barrier()
        pltpu.sync_copy(acc_sh.at[pl.ds(s * FRPT, FRPT)],
                        p_hbm.at[c].at[k].at[pl.ds(s * FRPT, FRPT)])

        @pl.when(s == NS - 1)
        def _tail():
            pltpu.sync_copy(
                acc_sh.at[pl.ds(NS * FRPT, FTAIL)],
                p_hbm.at[c].at[k].at[pl.ds(NS * FRPT, FTAIL)])
        plsc.subcore_barrier()


def _edges(g0, g1, g2, gsrc, sdst, zero_rows):
    return pl.kernel(
        _edge_body,
        out_type=jax.ShapeDtypeStruct((NC, K, N, D), jnp.float32),
        mesh=_sc_mesh(),
        scratch_types=[
            pltpu.VMEM((PPT, CH), jnp.int32),
            pltpu.VMEM((PPT, CH), jnp.int32),
            pltpu.VMEM((CH, D), jnp.float32),
            pltpu.VMEM((CH, D), jnp.float32),
            pltpu.VMEM((CH, D), jnp.float32),
            pltpu.VMEM((CH, D), jnp.float32),
            pltpu.SemaphoreType.DMA,
            pltpu.SemaphoreType.DMA,
            pltpu.SemaphoreType.DMA,
            pltpu.SemaphoreType.DMA,
            pltpu.SemaphoreType.DMA,
            pltpu.SemaphoreType.DMA,
            pltpu.SemaphoreType.DMA,
            pltpu.SemaphoreType.DMA,
            pltpu.VMEM_SHARED((NP, D), jnp.float32),
        ],
    )(g0, g1, g2, gsrc, sdst, zero_rows)


def _combine_body(p_ref, g0_ref, g1_ref, g2_ref, dinv_ref, out_ref):
    acc = jnp.zeros_like(out_ref[...])
    for k, g_ref in enumerate((g0_ref, g1_ref, g2_ref)):
        acc = acc + dinv_ref[:, k][:, None] * (
            p_ref[0, k] + p_ref[1, k] + g_ref[...])
    out_ref[...] = acc


def _combine(p, g0, g1, g2, dinv):
    B = 1000
    return pl.pallas_call(
        _combine_body,
        grid=(N // B,),
        in_specs=[
            pl.BlockSpec((NC, K, B, D), lambda i: (0, 0, i, 0)),
        ] + [pl.BlockSpec((B, D), lambda i: (i, 0))] * K + [
            pl.BlockSpec((B, K), lambda i: (i, 0)),
        ],
        out_specs=pl.BlockSpec((B, D), lambda i: (i, 0)),
        out_shape=jax.ShapeDtypeStruct((N, D), jnp.float32),
    )(p, g0, g1, g2, dinv)


def kernel(x, adj0, adj1, adj2, W0, W1, W2):
    adjs = [jnp.asarray(a, jnp.int32) for a in (adj0, adj1, adj2)]
    pad = E_PAD - E
    pad_src = (jnp.arange(pad, dtype=jnp.int32) * 977) % N
    pad_dst = N + (jnp.arange(pad, dtype=jnp.int32) % (NP - N))

    def r2(v):
        return v.reshape(-1, 128)

    gsrc = jnp.concatenate(
        [r2(adjs[0][0]), r2(pad_src), r2(adjs[1][0]), r2(pad_src),
         r2(adjs[2][0]), r2(pad_src)]).reshape(K, NCHUNK, CH)
    dcat = jnp.concatenate(
        [r2(adjs[0][1]), r2(pad_dst), r2(adjs[1][1]), r2(pad_dst),
         r2(adjs[2][1]), r2(pad_dst)])
    sdst = dcat.reshape(K, NCHUNK, CH)
    doffs = (jnp.arange(K, dtype=jnp.int32) * NP)[:, None]
    dd = (dcat.reshape(K, E_PAD) + doffs).reshape(DEG_ROWS, DCH)

    zero_deg = jnp.zeros((K * NP,), jnp.float32)
    zero_rows = jnp.zeros((NP, D), jnp.float32)
    wcat = jnp.concatenate([W0, W1, W2], axis=1)

    degs = _degrees(dd, zero_deg)
    deg_t = (degs.reshape(NC, K, NP)[:, :, :N]
             .transpose(2, 0, 1).reshape(N, NC * K))
    g0, g1, g2, dinv = _scale(x, wcat, deg_t)
    p = _edges(g0, g1, g2, gsrc, sdst, zero_rows)
    return _combine(p, g0, g1, g2, dinv)


# speedup vs baseline: 1.1798x; 1.0616x over previous
import jax
import jax.numpy as jnp
from jax import lax
from jax.experimental import pallas as pl
from jax.experimental.pallas import tpu as pltpu
from jax.experimental.pallas import tpu_sc as plsc

N = 10000
NP = 10240
E = 320000
D = 128
K = 3
CH = 64
DCH = 128
E_PAD = 327680
NCHUNK = E_PAD // CH
NC, NS = 2, 16
CPS = NCHUNK // NC
CPT = CPS // NS
NPHASE = 4
PPT = CPT // NPHASE
DEG_ROWS = K * E_PAD // DCH
DEG_RPT = DEG_ROWS // (NC * NS)
RPT = NP // NS
FRPT = 624
FTAIL = N - NS * FRPT


def _sc_mesh():
    return plsc.VectorSubcoreMesh(core_axis_name="c", subcore_axis_name="s")


def _deg_body(dd_hbm, zero_hbm, out_hbm, idx_v, ones_v, dsem, acc_sh):
    c = lax.axis_index("c")
    s = lax.axis_index("s")
    wid = c * NS + s

    @pl.when(s == 0)
    def _init():
        pltpu.sync_copy(zero_hbm, acc_sh)

    for i in range(8):
        ones_v[pl.ds(i * 16, 16)] = jnp.ones((16,), jnp.float32)
    pltpu.sync_copy(dd_hbm.at[pl.ds(wid * DEG_RPT, DEG_RPT)], idx_v)
    plsc.subcore_barrier()

    W = 16
    for b in range(W):
        pltpu.async_copy(ones_v, acc_sh.at[idx_v.at[b]], dsem, add=True)

    def step(j, carry):
        pltpu.make_async_copy(ones_v, acc_sh.at[idx_v.at[0]], dsem).wait()
        pltpu.async_copy(ones_v, acc_sh.at[idx_v.at[j + W]], dsem, add=True)
        return carry

    lax.fori_loop(0, DEG_RPT - W, step, 0)
    for b in range(W):
        pltpu.make_async_copy(ones_v, acc_sh.at[idx_v.at[0]], dsem).wait()
    plsc.subcore_barrier()

    @pl.when(s == 0)
    def _flush():
        pltpu.sync_copy(acc_sh, out_hbm.at[c])


def _degrees(dd, zero_deg):
    return pl.kernel(
        _deg_body,
        out_type=jax.ShapeDtypeStruct((NC, K * NP), jnp.float32),
        mesh=_sc_mesh(),
        scratch_types=[
            pltpu.VMEM((DEG_RPT, DCH), jnp.int32),
            pltpu.VMEM((DCH,), jnp.float32),
            pltpu.SemaphoreType.DMA,
            pltpu.VMEM_SHARED((K * NP,), jnp.float32),
        ],
    )(dd, zero_deg)


def _scale_body(x_ref, w_ref, deg_ref, g_ref, dinv_ref):
    deg = deg_ref[...]
    degsum = deg[:, :K] + deg[:, K:] + 1.0
    dinv = lax.rsqrt(jnp.maximum(degsum, 1e-12))
    h = jnp.dot(x_ref[...], w_ref[...],
                preferred_element_type=jnp.float32)
    for k in range(K):
        g_ref[k] = h[:, k * D:(k + 1) * D] * dinv[:, k][:, None]
    dinv_ref[...] = dinv


def _scale(x, wcat, deg_t):
    B = 2000
    return pl.pallas_call(
        _scale_body,
        grid=(N // B,),
        in_specs=[
            pl.BlockSpec((B, D), lambda i: (i, 0)),
            pl.BlockSpec((D, K * D), lambda i: (0, 0)),
            pl.BlockSpec((B, NC * K), lambda i: (i, 0)),
        ],
        out_specs=[
            pl.BlockSpec((K, B, D), lambda i: (0, i, 0)),
            pl.BlockSpec((B, K), lambda i: (i, 0)),
        ],
        out_shape=[
            jax.ShapeDtypeStruct((K, N, D), jnp.float32),
            jax.ShapeDtypeStruct((N, K), jnp.float32),
        ],
    )(x, wcat, deg_t)


NBUF = 4


def _edge_body(g_hbm, gsrc_hbm, sdst_hbm, zero_hbm, p_hbm,
               src_v, dst_v, r0, r1, r2, r3, g0, g1, g2, g3,
               s0, s1, s2, s3, acc_sh):
    rows = (r0, r1, r2, r3)
    gsems = (g0, g1, g2, g3)
    ssems = (s0, s1, s2, s3)
    c = lax.axis_index("c")
    s = lax.axis_index("s")
    for k in range(K):
        pltpu.sync_copy(zero_hbm.at[pl.ds(s * RPT, RPT)],
                        acc_sh.at[pl.ds(s * RPT, RPT)])
        for p in range(NPHASE):
            base = c * CPS + s * CPT + p * PPT
            pltpu.sync_copy(gsrc_hbm.at[k].at[pl.ds(base, PPT)], src_v)
            pltpu.sync_copy(sdst_hbm.at[k].at[pl.ds(base, PPT)], dst_v)
            if p == 0:
                plsc.subcore_barrier()

            for b in range(NBUF - 1):
                pltpu.async_copy(g_hbm.at[src_v.at[b]], rows[b], gsems[b])

            def rnd(jj, carry):
                for b in range(NBUF):
                    j = NBUF * jj + b
                    pltpu.make_async_copy(
                        g_hbm.at[src_v.at[j]], rows[b], gsems[b]).wait()
                    pltpu.async_copy(
                        rows[b], acc_sh.at[dst_v.at[j]], ssems[b], add=True)
                    nxt = j + NBUF - 1
                    bn = (b + NBUF - 1) % NBUF

                    @pl.when(jnp.logical_and(j >= 1, nxt < PPT))
                    def _wait_prev():
                        pltpu.make_async_copy(
                            rows[bn], acc_sh.at[dst_v.at[j]],
                            ssems[bn]).wait()

                    @pl.when(nxt < PPT)
                    def _issue():
                        pltpu.async_copy(
                            g_hbm.at[src_v.at[nxt]], rows[bn], gsems[bn])
                return carry

            lax.fori_loop(0, PPT // NBUF, rnd, 0)
            for b in range(NBUF):
                pltpu.make_async_copy(
                    rows[b], acc_sh.at[dst_v.at[0]], ssems[b]).wait()
        plsc.subcore_barrier()
        pltpu.sync_copy(acc_sh.at[pl.ds(s * FRPT, FRPT)],
                        p_hbm.at[c].at[k].at[pl.ds(s * FRPT, FRPT)])

        @pl.when(s == NS - 1)
        def _tail():
            pltpu.sync_copy(
                acc_sh.at[pl.ds(NS * FRPT, FTAIL)],
                p_hbm.at[c].at[k].at[pl.ds(NS * FRPT, FTAIL)])
        plsc.subcore_barrier()


def _edges(g_flat, gsrc, sdst, zero_rows):
    return pl.kernel(
        _edge_body,
        out_type=jax.ShapeDtypeStruct((NC, K, N, D), jnp.float32),
        mesh=_sc_mesh(),
        scratch_types=[
            pltpu.VMEM((PPT, CH), jnp.int32),
            pltpu.VMEM((PPT, CH), jnp.int32),
            pltpu.VMEM((CH, D), jnp.float32),
            pltpu.VMEM((CH, D), jnp.float32),
            pltpu.VMEM((CH, D), jnp.float32),
            pltpu.VMEM((CH, D), jnp.float32),
            pltpu.SemaphoreType.DMA,
            pltpu.SemaphoreType.DMA,
            pltpu.SemaphoreType.DMA,
            pltpu.SemaphoreType.DMA,
            pltpu.SemaphoreType.DMA,
            pltpu.SemaphoreType.DMA,
            pltpu.SemaphoreType.DMA,
            pltpu.SemaphoreType.DMA,
            pltpu.VMEM_SHARED((NP, D), jnp.float32),
        ],
    )(g_flat, gsrc, sdst, zero_rows)


def _combine_body(p_ref, g_ref, dinv_ref, out_ref):
    acc = jnp.zeros_like(out_ref[...])
    for k in range(K):
        acc = acc + dinv_ref[:, k][:, None] * (
            p_ref[0, k] + p_ref[1, k] + g_ref[k])
    out_ref[...] = acc


def _combine(p, g, dinv):
    B = 2000
    return pl.pallas_call(
        _combine_body,
        grid=(N // B,),
        in_specs=[
            pl.BlockSpec((NC, K, B, D), lambda i: (0, 0, i, 0)),
            pl.BlockSpec((K, B, D), lambda i: (0, i, 0)),
            pl.BlockSpec((B, K), lambda i: (i, 0)),
        ],
        out_specs=pl.BlockSpec((B, D), lambda i: (i, 0)),
        out_shape=jax.ShapeDtypeStruct((N, D), jnp.float32),
    )(p, g, dinv)


def kernel(x, adj0, adj1, adj2, W0, W1, W2):
    adjs = [jnp.asarray(a, jnp.int32) for a in (adj0, adj1, adj2)]
    pad = E_PAD - E
    pad_src = (jnp.arange(pad, dtype=jnp.int32) * 977) % N
    pad_dst = N + (jnp.arange(pad, dtype=jnp.int32) % (NP - N))

    src3 = jnp.stack([a[0] for a in adjs])
    dst3 = jnp.stack([a[1] for a in adjs])
    offs = (jnp.arange(K, dtype=jnp.int32) * N)[:, None]
    gsrc = (jnp.concatenate(
        [src3, jnp.broadcast_to(pad_src, (K, pad))], axis=1) + offs
            ).reshape(K, NCHUNK, CH)
    dcat = jnp.concatenate(
        [dst3, jnp.broadcast_to(pad_dst, (K, pad))], axis=1)
    sdst = dcat.reshape(K, NCHUNK, CH)
    doffs = (jnp.arange(K, dtype=jnp.int32) * NP)[:, None]
    dd = (dcat + doffs).reshape(DEG_ROWS, DCH)

    zero_deg = jnp.zeros((K * NP,), jnp.float32)
    zero_rows = jnp.zeros((NP, D), jnp.float32)
    wcat = jnp.concatenate([W0, W1, W2], axis=1)

    degs = _degrees(dd, zero_deg)
    deg_t = (degs.reshape(NC, K, NP)[:, :, :N]
             .transpose(2, 0, 1).reshape(N, NC * K))
    g, dinv = _scale(x, wcat, deg_t)
    p = _edges(g.reshape(K * N, D), gsrc, sdst, zero_rows)
    return _combine(p, g, dinv)


# speedup vs baseline: 1.1920x; 1.0104x over previous
import jax
import jax.numpy as jnp
from jax import lax
from jax.experimental import pallas as pl
from jax.experimental.pallas import tpu as pltpu
from jax.experimental.pallas import tpu_sc as plsc

N = 10000
NP = 10240
E = 320000
D = 128
K = 3
CH = 64
DCH = 128
E_PAD = 327680
NCHUNK = E_PAD // CH
NC, NS = 2, 16
CPS = NCHUNK // NC
CPT = CPS // NS
NPHASE = 4
PPT = CPT // NPHASE
DEG_ROWS = K * E_PAD // DCH
DEG_RPT = DEG_ROWS // (NC * NS)
RPT = NP // NS
FRPT = 624
FTAIL = N - NS * FRPT


def _sc_mesh():
    return plsc.VectorSubcoreMesh(core_axis_name="c", subcore_axis_name="s")


def _deg_body(dd_hbm, zero_hbm, out_hbm, idx_v, ones_v, dsem, acc_sh):
    c = lax.axis_index("c")
    s = lax.axis_index("s")
    wid = c * NS + s

    @pl.when(s == 0)
    def _init():
        pltpu.sync_copy(zero_hbm, acc_sh)

    for i in range(8):
        ones_v[pl.ds(i * 16, 16)] = jnp.ones((16,), jnp.float32)
    pltpu.sync_copy(dd_hbm.at[pl.ds(wid * DEG_RPT, DEG_RPT)], idx_v)
    plsc.subcore_barrier()

    W = 16
    for b in range(W):
        pltpu.async_copy(ones_v, acc_sh.at[idx_v.at[b]], dsem, add=True)

    def step(j, carry):
        pltpu.make_async_copy(ones_v, acc_sh.at[idx_v.at[0]], dsem).wait()
        pltpu.async_copy(ones_v, acc_sh.at[idx_v.at[j + W]], dsem, add=True)
        return carry

    lax.fori_loop(0, DEG_RPT - W, step, 0)
    for b in range(W):
        pltpu.make_async_copy(ones_v, acc_sh.at[idx_v.at[0]], dsem).wait()
    plsc.subcore_barrier()

    @pl.when(s == 0)
    def _flush():
        pltpu.sync_copy(acc_sh, out_hbm.at[c])


def _degrees(dd, zero_deg):
    return pl.kernel(
        _deg_body,
        out_type=jax.ShapeDtypeStruct((NC, K * NP), jnp.float32),
        mesh=_sc_mesh(),
        scratch_types=[
            pltpu.VMEM((DEG_RPT, DCH), jnp.int32),
            pltpu.VMEM((DCH,), jnp.float32),
            pltpu.SemaphoreType.DMA,
            pltpu.VMEM_SHARED((K * NP,), jnp.float32),
        ],
    )(dd, zero_deg)


def _scale_body(x_ref, w_ref, deg_ref, g_ref, dinv_ref):
    deg = deg_ref[...]
    degsum = deg[:, :K] + deg[:, K:] + 1.0
    dinv = lax.rsqrt(jnp.maximum(degsum, 1e-12))
    h = jnp.dot(x_ref[...], w_ref[...],
                preferred_element_type=jnp.float32)
    for k in range(K):
        g_ref[k] = h[:, k * D:(k + 1) * D] * dinv[:, k][:, None]
    dinv_ref[...] = dinv


def _scale(x, wcat, deg_t):
    B = 2000
    return pl.pallas_call(
        _scale_body,
        grid=(N // B,),
        in_specs=[
            pl.BlockSpec((B, D), lambda i: (i, 0)),
            pl.BlockSpec((D, K * D), lambda i: (0, 0)),
            pl.BlockSpec((B, NC * K), lambda i: (i, 0)),
        ],
        out_specs=[
            pl.BlockSpec((K, B, D), lambda i: (0, i, 0)),
            pl.BlockSpec((B, K), lambda i: (i, 0)),
        ],
        out_shape=[
            jax.ShapeDtypeStruct((K, N, D), jnp.float32),
            jax.ShapeDtypeStruct((N, K), jnp.float32),
        ],
    )(x, wcat, deg_t)


NBUF = 4


def _edge_body(g_hbm, gsrc_hbm, sdst_hbm, zero_hbm, p_hbm,
               src_v, dst_v, r0, r1, r2, r3, g0, g1, g2, g3,
               s0, s1, s2, s3, acc_sh):
    rows = (r0, r1, r2, r3)
    gsems = (g0, g1, g2, g3)
    ssems = (s0, s1, s2, s3)
    c = lax.axis_index("c")
    s = lax.axis_index("s")
    for k in range(K):
        pltpu.async_copy(zero_hbm.at[pl.ds(s * RPT, RPT)],
                         acc_sh.at[pl.ds(s * RPT, RPT)], gsems[0])
        for p in range(NPHASE):
            base = c * CPS + s * CPT + p * PPT
            pltpu.sync_copy(gsrc_hbm.at[k].at[pl.ds(base, PPT)], src_v)
            pltpu.sync_copy(sdst_hbm.at[k].at[pl.ds(base, PPT)], dst_v)
            if p == 0:
                pltpu.make_async_copy(
                    zero_hbm.at[pl.ds(s * RPT, RPT)],
                    acc_sh.at[pl.ds(s * RPT, RPT)], gsems[0]).wait()
                plsc.subcore_barrier()

            for b in range(NBUF - 1):
                pltpu.async_copy(g_hbm.at[src_v.at[b]], rows[b], gsems[b])

            def rnd(jj, carry):
                for b in range(NBUF):
                    j = NBUF * jj + b
                    pltpu.make_async_copy(
                        g_hbm.at[src_v.at[j]], rows[b], gsems[b]).wait()
                    pltpu.async_copy(
                        rows[b], acc_sh.at[dst_v.at[j]], ssems[b], add=True)
                    nxt = j + NBUF - 1
                    bn = (b + NBUF - 1) % NBUF

                    @pl.when(jnp.logical_and(j >= 1, nxt < PPT))
                    def _wait_prev():
                        pltpu.make_async_copy(
                            rows[bn], acc_sh.at[dst_v.at[j]],
                            ssems[bn]).wait()

                    @pl.when(nxt < PPT)
                    def _issue():
                        pltpu.async_copy(
                            g_hbm.at[src_v.at[nxt]], rows[bn], gsems[bn])
                return carry

            lax.fori_loop(0, PPT // NBUF, rnd, 0)
            for b in range(NBUF):
                pltpu.make_async_copy(
                    rows[b], acc_sh.at[dst_v.at[0]], ssems[b]).wait()
        plsc.subcore_barrier()
        pltpu.sync_copy(acc_sh.at[pl.ds(s * FRPT, FRPT)],
                        p_hbm.at[c].at[k].at[pl.ds(s * FRPT, FRPT)])

        @pl.when(s == NS - 1)
        def _tail():
            pltpu.sync_copy(
                acc_sh.at[pl.ds(NS * FRPT, FTAIL)],
                p_hbm.at[c].at[k].at[pl.ds(NS * FRPT, FTAIL)])
        plsc.subcore_barrier()


def _edges(g_flat, gsrc, sdst, zero_rows):
    return pl.kernel(
        _edge_body,
        out_type=jax.ShapeDtypeStruct((NC, K, N, D), jnp.float32),
        mesh=_sc_mesh(),
        scratch_types=[
            pltpu.VMEM((PPT, CH), jnp.int32),
            pltpu.VMEM((PPT, CH), jnp.int32),
            pltpu.VMEM((CH, D), jnp.float32),
            pltpu.VMEM((CH, D), jnp.float32),
            pltpu.VMEM((CH, D), jnp.float32),
            pltpu.VMEM((CH, D), jnp.float32),
            pltpu.SemaphoreType.DMA,
            pltpu.SemaphoreType.DMA,
            pltpu.SemaphoreType.DMA,
            pltpu.SemaphoreType.DMA,
            pltpu.SemaphoreType.DMA,
            pltpu.SemaphoreType.DMA,
            pltpu.SemaphoreType.DMA,
            pltpu.SemaphoreType.DMA,
            pltpu.VMEM_SHARED((NP, D), jnp.float32),
        ],
    )(g_flat, gsrc, sdst, zero_rows)


def _combine_body(p_ref, g_ref, dinv_ref, out_ref):
    acc = jnp.zeros_like(out_ref[...])
    for k in range(K):
        acc = acc + dinv_ref[:, k][:, None] * (
            p_ref[0, k] + p_ref[1, k] + g_ref[k])
    out_ref[...] = acc


def _combine(p, g, dinv):
    B = 2000
    return pl.pallas_call(
        _combine_body,
        grid=(N // B,),
        in_specs=[
            pl.BlockSpec((NC, K, B, D), lambda i: (0, 0, i, 0)),
            pl.BlockSpec((K, B, D), lambda i: (0, i, 0)),
            pl.BlockSpec((B, K), lambda i: (i, 0)),
        ],
        out_specs=pl.BlockSpec((B, D), lambda i: (i, 0)),
        out_shape=jax.ShapeDtypeStruct((N, D), jnp.float32),
    )(p, g, dinv)


def kernel(x, adj0, adj1, adj2, W0, W1, W2):
    adjs = [jnp.asarray(a, jnp.int32) for a in (adj0, adj1, adj2)]
    pad = E_PAD - E
    pad_src = (jnp.arange(pad, dtype=jnp.int32) * 977) % N
    pad_dst = N + (jnp.arange(pad, dtype=jnp.int32) % (NP - N))

    src3 = jnp.stack([a[0] for a in adjs])
    dst3 = jnp.stack([a[1] for a in adjs])
    offs = (jnp.arange(K, dtype=jnp.int32) * N)[:, None]
    gsrc = (jnp.concatenate(
        [src3, jnp.broadcast_to(pad_src, (K, pad))], axis=1) + offs
            ).reshape(K, NCHUNK, CH)
    dcat = jnp.concatenate(
        [dst3, jnp.broadcast_to(pad_dst, (K, pad))], axis=1)
    sdst = dcat.reshape(K, NCHUNK, CH)
    doffs = (jnp.arange(K, dtype=jnp.int32) * NP)[:, None]
    dd = (dcat + doffs).reshape(DEG_ROWS, DCH)

    zero_deg = jnp.zeros((K * NP,), jnp.float32)
    zero_rows = jnp.zeros((NP, D), jnp.float32)
    wcat = jnp.concatenate([W0, W1, W2], axis=1)

    degs = _degrees(dd, zero_deg)
    deg_t = (degs.reshape(NC, K, NP)[:, :, :N]
             .transpose(2, 0, 1).reshape(N, NC * K))
    g, dinv = _scale(x, wcat, deg_t)
    p = _edges(g.reshape(K * N, D), gsrc, sdst, zero_rows)
    return _combine(p, g, dinv)
